# Initial kernel scaffold; baseline (speedup 1.0000x reference)
import jax
import jax.numpy as jnp
from jax.experimental import pallas as pl


def kernel(x_feat, x_emb, edge_index, Wl1, bl1, Wr1, We1, be1, Wl2, bl2, Wr2, We2, be2):
    raise NotImplementedError("write your pallas kernel here")


# speedup vs baseline: 6.4721x; 6.4721x over previous
import functools

import jax
import jax.numpy as jnp
from jax import lax
from jax.experimental import pallas as pl
from jax.experimental.pallas import tpu as pltpu
from jax.experimental.pallas import tpu_sc as plsc

N = 10000
E = 320000
F = 64

NC = 2
NS = 16
NW = NC * NS
CHUNK = 128
NCH = -(-E // (NW * CHUNK))
EPAD = NW * NCH * CHUNK
NP = 10240
RPT = NP // NS


def _sc_body(src_h, dst_h, p_h, zb_h, zc_h, one_h, agg_o, cnt_o,
             srcv, dstv, rowsv, acc_s, cnt_s, onesv, sem):
    c = lax.axis_index("c")
    s = lax.axis_index("s")
    wid = c * NS + s

    pltpu.sync_copy(src_h.at[wid], srcv)
    pltpu.sync_copy(dst_h.at[wid], dstv)
    pltpu.sync_copy(one_h, onesv)

    pltpu.sync_copy(zb_h, acc_s.at[pl.ds(s * RPT, RPT)])
    pltpu.sync_copy(zc_h, cnt_s.at[pl.ds(s * RPT, RPT)])
    plsc.subcore_barrier()

    def step(j, carry):
        pltpu.async_copy(p_h.at[srcv.at[j]], rowsv, sem).wait()
        pltpu.sync_copy(rowsv, acc_s.at[dstv.at[j]], add=True)
        pltpu.sync_copy(onesv, cnt_s.at[dstv.at[j]], add=True)
        return carry

    lax.fori_loop(0, NCH, step, 0)
    plsc.subcore_barrier()

    @pl.when(s < NS - 1)
    def _():
        pltpu.sync_copy(acc_s.at[pl.ds(s * RPT, RPT)],
                        agg_o.at[c, pl.ds(s * RPT, RPT)])
        pltpu.sync_copy(cnt_s.at[pl.ds(s * RPT, RPT)],
                        cnt_o.at[c, pl.ds(s * RPT, RPT)])

    @pl.when(s == NS - 1)
    def _():
        last = N - (NS - 1) * RPT
        pltpu.sync_copy(acc_s.at[pl.ds((NS - 1) * RPT, last)],
                        agg_o.at[c, pl.ds((NS - 1) * RPT, last)])
        pltpu.sync_copy(cnt_s.at[pl.ds((NS - 1) * RPT, last)],
                        cnt_o.at[c, pl.ds((NS - 1) * RPT, last)])


_sc_segsum = pl.kernel(
    _sc_body,
    out_type=(jax.ShapeDtypeStruct((NC, N, F), jnp.float32),
              jax.ShapeDtypeStruct((NC, N), jnp.float32)),
    mesh=plsc.VectorSubcoreMesh(core_axis_name="c", subcore_axis_name="s"),
    scratch_types=[
        pltpu.VMEM((NCH, CHUNK), jnp.int32),
        pltpu.VMEM((NCH, CHUNK), jnp.int32),
        pltpu.VMEM((CHUNK, F), jnp.float32),
        pltpu.VMEM_SHARED((NP, F), jnp.float32),
        pltpu.VMEM_SHARED((NP,), jnp.float32),
        pltpu.VMEM((CHUNK,), jnp.float32),
        pltpu.SemaphoreType.DMA,
    ],
    compiler_params=pltpu.CompilerParams(use_tc_tiling_on_sc=False),
)


BLK = 1000


def _tc1_body(x_r, xe_r, wl_r, wr_r, we_r, be_r, bl_r, p1_r, base1_r, e_r):
    x = x_r[...]
    xe = xe_r[...]
    e0 = (xe[:, 0:64] + xe[:, 64:128] + xe[:, 128:192] + xe[:, 192:256]) * 0.25
    e1 = jnp.dot(e0, we_r[...], preferred_element_type=jnp.float32) + be_r[...]
    p1_r[...] = jnp.dot(x, wl_r[...], preferred_element_type=jnp.float32)
    base1_r[...] = (jnp.dot(x, wr_r[...], preferred_element_type=jnp.float32)
                    + e1 + bl_r[...])
    e_r[...] = e1


def _tc2_body(agg_r, cnt_r, base1_r, e_r, wl2_r, wr2_r, we2_r, b2_r,
              p2_r, base2_r):
    agg = agg_r[0] + agg_r[1]
    csum = cnt_r[:, 0:1] + cnt_r[:, 1:2]
    inv = 1.0 / jnp.maximum(csum, 1.0)
    x = jnp.maximum(agg * inv + base1_r[...], 0.0)
    p2_r[...] = jnp.dot(x, wl2_r[...], preferred_element_type=jnp.float32)
    base2_r[...] = (jnp.dot(x, wr2_r[...], preferred_element_type=jnp.float32)
                    + jnp.dot(e_r[...], we2_r[...],
                              preferred_element_type=jnp.float32)
                    + b2_r[...])


def _tc3_body(agg_r, cnt_r, base2_r, out_r):
    agg = agg_r[0] + agg_r[1]
    csum = cnt_r[:, 0:1] + cnt_r[:, 1:2]
    inv = 1.0 / jnp.maximum(csum, 1.0)
    out_r[...] = agg * inv + base2_r[...]


def _row_spec(width):
    return pl.BlockSpec((BLK, width), lambda i: (i, 0))


def _full_spec(shape):
    nd = len(shape)
    return pl.BlockSpec(shape, lambda i: (0,) * nd)


_tc1 = pl.pallas_call(
    _tc1_body,
    grid=(N // BLK,),
    in_specs=[_row_spec(128), _row_spec(256), _full_spec((128, F)),
              _full_spec((128, F)), _full_spec((F, F)), _full_spec((1, F)),
              _full_spec((1, F))],
    out_specs=[_row_spec(F), _row_spec(F), _row_spec(F)],
    out_shape=[jax.ShapeDtypeStruct((N, F), jnp.float32)] * 3,
)

_agg_spec = pl.BlockSpec((NC, BLK, F), lambda i: (0, i, 0))
_cnt_spec = pl.BlockSpec((BLK, NC), lambda i: (i, 0))

_tc2 = pl.pallas_call(
    _tc2_body,
    grid=(N // BLK,),
    in_specs=[_agg_spec, _cnt_spec, _row_spec(F), _row_spec(F),
              _full_spec((F, F)), _full_spec((F, F)), _full_spec((F, F)),
              _full_spec((1, F))],
    out_specs=[_row_spec(F), _row_spec(F)],
    out_shape=[jax.ShapeDtypeStruct((N, F), jnp.float32)] * 2,
)

_tc3 = pl.pallas_call(
    _tc3_body,
    grid=(N // BLK,),
    in_specs=[_agg_spec, _cnt_spec, _row_spec(F)],
    out_specs=_row_spec(F),
    out_shape=jax.ShapeDtypeStruct((N, F), jnp.float32),
)


def kernel(x_feat, x_emb, edge_index, Wl1, bl1, Wr1, We1, be1,
           Wl2, bl2, Wr2, We2, be2):
    src = edge_index[0]
    dst = edge_index[1]
    pad = EPAD - E
    src_p = jnp.concatenate([src, jnp.zeros((pad,), jnp.int32)])
    dst_p = jnp.concatenate([dst, jnp.full((pad,), N, jnp.int32)])
    src3 = src_p.reshape(NW, NCH, CHUNK)
    dst3 = dst_p.reshape(NW, NCH, CHUNK)
    xe = x_emb.reshape(N, x_emb.shape[1] * x_emb.shape[2])
    b1e = be1.reshape(1, F)
    b1l = bl1.reshape(1, F)
    b2 = (bl2 + be2).reshape(1, F)
    zb = jnp.zeros((RPT, F), jnp.float32)
    zc = jnp.zeros((RPT,), jnp.float32)
    ones = jnp.ones((CHUNK,), jnp.float32)

    p1, base1, e = _tc1(x_feat, xe, Wl1, Wr1, We1, b1e, b1l)

    agg1, cnt = _sc_segsum(src3, dst3, p1, zb, zc, ones)
    cnt_t = cnt.T

    p2, base2 = _tc2(agg1, cnt_t, base1, e, Wl2, Wr2, We2, b2)

    agg2, _ = _sc_segsum(src3, dst3, p2, zb, zc, ones)

    return _tc3(agg2, cnt_t, base2)
